# Initial kernel scaffold; baseline (speedup 1.0000x reference)
import jax
import jax.numpy as jnp
from jax.experimental import pallas as pl


def kernel(inference, ground_truth):
    raise NotImplementedError("write your pallas kernel here")


# speedup vs baseline: 1.8198x; 1.8198x over previous
import jax
import jax.numpy as jnp
from jax.experimental import pallas as pl

B = 16384
C = 10


def _body(x_ref, g_ref, soft_ref, ohe_ref):
    e = jnp.exp(x_ref[...])
    soft_ref[...] = e / jnp.sum(e)
    cls = jax.lax.broadcasted_iota(jnp.int32, (B, C), 1)
    ohe_ref[...] = (g_ref[...] == cls).astype(jnp.float32)


def kernel(inference, ground_truth):
    gt2 = ground_truth.astype(jnp.int32).reshape(B, 1)
    soft, ohe = pl.pallas_call(
        _body,
        out_shape=(
            jax.ShapeDtypeStruct((B, C), jnp.float32),
            jax.ShapeDtypeStruct((B, C), jnp.float32),
        ),
    )(inference, gt2)
    return (soft, ohe)
